# Initial kernel scaffold; baseline (speedup 1.0000x reference)
import jax
import jax.numpy as jnp
from jax.experimental import pallas as pl


def kernel(user, pos, neg, mf_user_w, mf_item_w, mlp_user_w, mlp_item_w, train_label):
    raise NotImplementedError("write your pallas kernel here")


# speedup vs baseline: 4.1556x; 4.1556x over previous
import functools

import jax
import jax.numpy as jnp
from jax import lax
from jax.experimental import pallas as pl
from jax.experimental.pallas import tpu as pltpu
from jax.experimental.pallas import tpu_sc as plsc

NUM_USERS = 100000
NUM_ITEMS_P1 = 1001
DIM = 128
B = 4096

NC = 2
NS = 16
NW = NC * NS
BPW = B // NW
LCHUNK = 8
NCHUNK = BPW // LCHUNK


def _sc_gather_body(user_hbm, pos_hbm, neg_hbm,
                    mf_user_w, mlp_user_w, mf_item_w, mlp_item_w,
                    train_label,
                    mf_user_out, mlp_user_out,
                    mf_pos_out, mf_neg_out, mlp_pos_out, mlp_neg_out,
                    label_out,
                    idx_u, idx_p, idx_n,
                    embs, lbl0, lbl1,
                    sem_emb, sem_l0, sem_l1):
  wid = lax.axis_index("s") * NC + lax.axis_index("c")
  base = wid * BPW

  pltpu.sync_copy(user_hbm.at[pl.ds(base, BPW)], idx_u)
  pltpu.sync_copy(pos_hbm.at[pl.ds(base, BPW)], idx_p)
  pltpu.sync_copy(neg_hbm.at[pl.ds(base, BPW)], idx_n)

  jobs = ((mf_user_w, idx_u, mf_user_out),
          (mlp_user_w, idx_u, mlp_user_out),
          (mf_item_w, idx_p, mf_pos_out),
          (mf_item_w, idx_n, mf_neg_out),
          (mlp_item_w, idx_p, mlp_pos_out),
          (mlp_item_w, idx_n, mlp_neg_out))
  descs = []
  for j, (table, idx, _) in enumerate(jobs):
    d = pltpu.make_async_copy(table.at[idx], embs.at[j], sem_emb)
    d.start()
    descs.append(d)

  lbl_bufs = (lbl0, lbl1)
  lbl_sems = (sem_l0, sem_l1)

  def _fire_label(c):
    vbase = (c * LCHUNK // 16) * 16
    lane0 = c * LCHUNK - vbase
    v = idx_u[pl.ds(vbase, 16)]
    ds = []
    for i in range(LCHUNK):
      u = v[lane0 + i]
      d = pltpu.make_async_copy(train_label.at[u],
                                lbl_bufs[c % 2].at[i], lbl_sems[c % 2])
      d.start()
      ds.append(d)
    return ds

  lbl_descs = [None] * NCHUNK
  lbl_descs[0] = _fire_label(0)

  for d in descs:
    d.wait()
  for j, (_, _, out) in enumerate(jobs):
    pltpu.sync_copy(embs.at[j], out.at[pl.ds(base, BPW)])

  for c in range(NCHUNK):
    for d in lbl_descs[c]:
      d.wait()
    if c + 1 < NCHUNK:
      lbl_descs[c + 1] = _fire_label(c + 1)
    pltpu.sync_copy(lbl_bufs[c % 2],
                    label_out.at[pl.ds(base + c * LCHUNK, LCHUNK)])


@functools.partial(jax.jit, static_argnames=())
def _sc_gather(user, pos, neg, mf_user_w, mlp_user_w, mf_item_w, mlp_item_w,
               train_label):
  mesh = plsc.VectorSubcoreMesh(core_axis_name="c", subcore_axis_name="s",
                                num_cores=NC, num_subcores=NS)
  f32 = jnp.float32
  out_type = [jax.ShapeDtypeStruct((B, DIM), f32) for _ in range(6)]
  out_type.append(jax.ShapeDtypeStruct((B, NUM_ITEMS_P1), f32))
  kern = functools.partial(
      pl.kernel,
      out_type=out_type,
      mesh=mesh,
      scratch_types=[
          pltpu.VMEM((BPW,), jnp.int32),
          pltpu.VMEM((BPW,), jnp.int32),
          pltpu.VMEM((BPW,), jnp.int32),
          pltpu.VMEM((6, BPW, DIM), f32),
          pltpu.VMEM((LCHUNK, NUM_ITEMS_P1), f32),
          pltpu.VMEM((LCHUNK, NUM_ITEMS_P1), f32),
          pltpu.SemaphoreType.DMA,
          pltpu.SemaphoreType.DMA,
          pltpu.SemaphoreType.DMA,
      ],
  )(_sc_gather_body)
  return kern(user, pos, neg, mf_user_w, mlp_user_w, mf_item_w, mlp_item_w,
              train_label)


BR = 512


def _community_body(lbl_ref, mfw_ref, mlpw_ref, mf_out, mlp_out):
  lbl = lbl_ref[...]
  inv = 1.0 / jnp.sum(lbl, axis=1, keepdims=True)
  mf_out[...] = jnp.dot(lbl, mfw_ref[...],
                        preferred_element_type=jnp.float32) * inv
  mlp_out[...] = jnp.dot(lbl, mlpw_ref[...],
                         preferred_element_type=jnp.float32) * inv


def _community(batch_label, mf_item_w, mlp_item_w):
  grid = (B // BR,)
  return pl.pallas_call(
      _community_body,
      grid=grid,
      in_specs=[
          pl.BlockSpec((BR, NUM_ITEMS_P1), lambda i: (i, 0)),
          pl.BlockSpec((NUM_ITEMS_P1, DIM), lambda i: (0, 0)),
          pl.BlockSpec((NUM_ITEMS_P1, DIM), lambda i: (0, 0)),
      ],
      out_specs=[
          pl.BlockSpec((BR, DIM), lambda i: (i, 0)),
          pl.BlockSpec((BR, DIM), lambda i: (i, 0)),
      ],
      out_shape=[
          jax.ShapeDtypeStruct((B, DIM), jnp.float32),
          jax.ShapeDtypeStruct((B, DIM), jnp.float32),
      ],
  )(batch_label, mf_item_w, mlp_item_w)


def kernel(user, pos, neg, mf_user_w, mf_item_w, mlp_user_w, mlp_item_w,
           train_label):
  user = user.astype(jnp.int32)
  pos = pos.astype(jnp.int32)
  neg = neg.astype(jnp.int32)
  (mf_user_emb, mlp_user_emb, mf_pos_emb, mf_neg_emb, mlp_pos_emb,
   mlp_neg_emb, batch_label) = _sc_gather(
       user, pos, neg, mf_user_w, mlp_user_w, mf_item_w, mlp_item_w,
       train_label)
  mf_pos_i_com, mlp_pos_i_com = _community(batch_label, mf_item_w, mlp_item_w)
  return (mf_user_emb, mf_pos_emb, mf_neg_emb, mf_pos_i_com,
          mlp_user_emb, mlp_pos_emb, mlp_neg_emb, mlp_pos_i_com)


# speedup vs baseline: 6.9113x; 1.6631x over previous
import functools

import jax
import jax.numpy as jnp
from jax import lax
from jax.experimental import pallas as pl
from jax.experimental.pallas import tpu as pltpu
from jax.experimental.pallas import tpu_sc as plsc

NUM_USERS = 100000
NUM_ITEMS_P1 = 1001
DIM = 128
B = 4096

NC = 2
NS = 16
NW = NC * NS
BPW = B // NW

BU = 1024
NAUG = 384


def _community_all_body(t_ref, w_ref, out_ref):
  tb = t_ref[...].astype(jnp.bfloat16)
  p = lax.dot_general(tb, w_ref[...],
                      (((0,), (0,)), ((), ())),
                      preferred_element_type=jnp.float32)
  num = p[:, 256:257]
  out_ref[...] = p[:, :256] / num


def _community_all(t, w_aug):
  grid = (pl.cdiv(NUM_USERS, BU),)
  return pl.pallas_call(
      _community_all_body,
      grid=grid,
      in_specs=[
          pl.BlockSpec((NUM_ITEMS_P1, BU), lambda i: (0, i)),
          pl.BlockSpec((NUM_ITEMS_P1, NAUG), lambda i: (0, 0)),
      ],
      out_specs=pl.BlockSpec((BU, 256), lambda i: (i, 0)),
      out_shape=jax.ShapeDtypeStruct((NUM_USERS, 256), jnp.float32),
  )(t, w_aug)


def _sc_embed_body(user_hbm, pos_hbm, neg_hbm,
                   mf_user_w, mlp_user_w, mf_item_w, mlp_item_w,
                   mf_user_out, mlp_user_out,
                   mf_pos_out, mf_neg_out, mlp_pos_out, mlp_neg_out,
                   idx_u, idx_p, idx_n, embs, sem_emb):
  wid = lax.axis_index("s") * NC + lax.axis_index("c")
  base = wid * BPW

  pltpu.sync_copy(user_hbm.at[pl.ds(base, BPW)], idx_u)
  pltpu.sync_copy(pos_hbm.at[pl.ds(base, BPW)], idx_p)
  pltpu.sync_copy(neg_hbm.at[pl.ds(base, BPW)], idx_n)

  jobs = ((mf_user_w, idx_u, mf_user_out),
          (mlp_user_w, idx_u, mlp_user_out),
          (mf_item_w, idx_p, mf_pos_out),
          (mf_item_w, idx_n, mf_neg_out),
          (mlp_item_w, idx_p, mlp_pos_out),
          (mlp_item_w, idx_n, mlp_neg_out))
  descs = []
  for j, (table, idx, _) in enumerate(jobs):
    d = pltpu.make_async_copy(table.at[idx], embs.at[j], sem_emb)
    d.start()
    descs.append(d)
  for d in descs:
    d.wait()
  for j, (_, _, out) in enumerate(jobs):
    pltpu.sync_copy(embs.at[j], out.at[pl.ds(base, BPW)])


def _sc_embed(user, pos, neg, mf_user_w, mlp_user_w, mf_item_w, mlp_item_w):
  mesh = plsc.VectorSubcoreMesh(core_axis_name="c", subcore_axis_name="s",
                                num_cores=NC, num_subcores=NS)
  f32 = jnp.float32
  kern = functools.partial(
      pl.kernel,
      out_type=[jax.ShapeDtypeStruct((B, DIM), f32) for _ in range(6)],
      mesh=mesh,
      scratch_types=[
          pltpu.VMEM((BPW,), jnp.int32),
          pltpu.VMEM((BPW,), jnp.int32),
          pltpu.VMEM((BPW,), jnp.int32),
          pltpu.VMEM((6, BPW, DIM), f32),
          pltpu.SemaphoreType.DMA,
      ],
  )(_sc_embed_body)
  return kern(user, pos, neg, mf_user_w, mlp_user_w, mf_item_w, mlp_item_w)


def _sc_comgather_body(user_hbm, p_hbm, com_out, idx_u, rows, sem):
  wid = lax.axis_index("s") * NC + lax.axis_index("c")
  base = wid * BPW
  pltpu.sync_copy(user_hbm.at[pl.ds(base, BPW)], idx_u)
  pltpu.async_copy(p_hbm.at[idx_u], rows, sem).wait()
  pltpu.sync_copy(rows, com_out.at[pl.ds(base, BPW)])


def _sc_comgather(user, p_norm):
  mesh = plsc.VectorSubcoreMesh(core_axis_name="c", subcore_axis_name="s",
                                num_cores=NC, num_subcores=NS)
  kern = functools.partial(
      pl.kernel,
      out_type=[jax.ShapeDtypeStruct((B, 256), jnp.float32)],
      mesh=mesh,
      scratch_types=[
          pltpu.VMEM((BPW,), jnp.int32),
          pltpu.VMEM((BPW, 256), jnp.float32),
          pltpu.SemaphoreType.DMA,
      ],
  )(_sc_comgather_body)
  (com,) = kern(user, p_norm)
  return com


def kernel(user, pos, neg, mf_user_w, mf_item_w, mlp_user_w, mlp_item_w,
           train_label):
  user = user.astype(jnp.int32)
  pos = pos.astype(jnp.int32)
  neg = neg.astype(jnp.int32)

  t = train_label.T
  w_aug = jnp.concatenate(
      [mf_item_w, mlp_item_w,
       jnp.ones((NUM_ITEMS_P1, 1), jnp.float32),
       jnp.zeros((NUM_ITEMS_P1, NAUG - 257), jnp.float32)],
      axis=1).astype(jnp.bfloat16)

  p_norm = _community_all(t, w_aug)

  (mf_user_emb, mlp_user_emb, mf_pos_emb, mf_neg_emb, mlp_pos_emb,
   mlp_neg_emb) = _sc_embed(user, pos, neg, mf_user_w, mlp_user_w,
                            mf_item_w, mlp_item_w)
  com = _sc_comgather(user, p_norm)
  mf_pos_i_com = com[:, :DIM]
  mlp_pos_i_com = com[:, DIM:]
  return (mf_user_emb, mf_pos_emb, mf_neg_emb, mf_pos_i_com,
          mlp_user_emb, mlp_pos_emb, mlp_neg_emb, mlp_pos_i_com)


# speedup vs baseline: 9.2607x; 1.3399x over previous
import functools

import jax
import jax.numpy as jnp
from jax import lax
from jax.experimental import pallas as pl
from jax.experimental.pallas import tpu as pltpu
from jax.experimental.pallas import tpu_sc as plsc

NUM_USERS = 100000
NUM_ITEMS_P1 = 1001
DIM = 128
B = 4096

NC = 2
NS = 16
NW = NC * NS
BPW = B // NW

BU = 6144
NAUG = 384


def _community_all_body(t_ref, wt_ref, out_ref):
  tb = t_ref[...].astype(jnp.bfloat16)
  p = lax.dot_general(wt_ref[...], tb,
                      (((1,), (0,)), ((), ())),
                      preferred_element_type=jnp.float32)
  num = p[256:257, :]
  pn = p[:256, :] / num
  pn16 = pn.T.astype(jnp.bfloat16)
  lo = lax.bitcast_convert_type(pn16[:, :128], jnp.uint16).astype(jnp.uint32)
  hi = lax.bitcast_convert_type(pn16[:, 128:], jnp.uint16).astype(jnp.uint32)
  out_ref[...] = lax.bitcast_convert_type(lo | (hi << 16), jnp.float32)


def _community_all(t, w_aug):
  grid = (pl.cdiv(NUM_USERS, BU),)
  return pl.pallas_call(
      _community_all_body,
      grid=grid,
      in_specs=[
          pl.BlockSpec((NUM_ITEMS_P1, BU), lambda i: (0, i)),
          pl.BlockSpec((NAUG, NUM_ITEMS_P1), lambda i: (0, 0)),
      ],
      out_specs=pl.BlockSpec((BU, 128), lambda i: (i, 0)),
      out_shape=jax.ShapeDtypeStruct((NUM_USERS, 128), jnp.float32),
      compiler_params=pltpu.CompilerParams(
          vmem_limit_bytes=100 * 1024 * 1024),
  )(t, w_aug)


def _sc_embed_body(user_hbm, pos_hbm, neg_hbm,
                   mf_user_w, mlp_user_w, mf_item_w, mlp_item_w,
                   mf_user_out, mlp_user_out,
                   mf_pos_out, mf_neg_out, mlp_pos_out, mlp_neg_out,
                   idx_u, idx_p, idx_n, embs, sem_emb):
  wid = lax.axis_index("s") * NC + lax.axis_index("c")
  base = wid * BPW

  pltpu.sync_copy(user_hbm.at[pl.ds(base, BPW)], idx_u)
  pltpu.sync_copy(pos_hbm.at[pl.ds(base, BPW)], idx_p)
  pltpu.sync_copy(neg_hbm.at[pl.ds(base, BPW)], idx_n)

  jobs = ((mf_user_w, idx_u, mf_user_out),
          (mlp_user_w, idx_u, mlp_user_out),
          (mf_item_w, idx_p, mf_pos_out),
          (mf_item_w, idx_n, mf_neg_out),
          (mlp_item_w, idx_p, mlp_pos_out),
          (mlp_item_w, idx_n, mlp_neg_out))
  descs = []
  for j, (table, idx, _) in enumerate(jobs):
    d = pltpu.make_async_copy(table.at[idx], embs.at[j], sem_emb)
    d.start()
    descs.append(d)
  for d in descs:
    d.wait()
  for j, (_, _, out) in enumerate(jobs):
    pltpu.sync_copy(embs.at[j], out.at[pl.ds(base, BPW)])


def _sc_embed(user, pos, neg, mf_user_w, mlp_user_w, mf_item_w, mlp_item_w):
  mesh = plsc.VectorSubcoreMesh(core_axis_name="c", subcore_axis_name="s",
                                num_cores=NC, num_subcores=NS)
  f32 = jnp.float32
  kern = functools.partial(
      pl.kernel,
      out_type=[jax.ShapeDtypeStruct((B, DIM), f32) for _ in range(6)],
      mesh=mesh,
      scratch_types=[
          pltpu.VMEM((BPW,), jnp.int32),
          pltpu.VMEM((BPW,), jnp.int32),
          pltpu.VMEM((BPW,), jnp.int32),
          pltpu.VMEM((6, BPW, DIM), f32),
          pltpu.SemaphoreType.DMA,
      ],
  )(_sc_embed_body)
  return kern(user, pos, neg, mf_user_w, mlp_user_w, mf_item_w, mlp_item_w)


def _sc_comgather_body(user_hbm, p_hbm, pk_out, idx_u, rows, sem):
  wid = lax.axis_index("s") * NC + lax.axis_index("c")
  base = wid * BPW
  pltpu.sync_copy(user_hbm.at[pl.ds(base, BPW)], idx_u)
  pltpu.async_copy(p_hbm.at[idx_u], rows, sem).wait()
  pltpu.sync_copy(rows, pk_out.at[pl.ds(base, BPW)])


def _sc_comgather(user, p_norm):
  mesh = plsc.VectorSubcoreMesh(core_axis_name="c", subcore_axis_name="s",
                                num_cores=NC, num_subcores=NS)
  kern = functools.partial(
      pl.kernel,
      out_type=jax.ShapeDtypeStruct((B, DIM), jnp.float32),
      mesh=mesh,
      scratch_types=[
          pltpu.VMEM((BPW,), jnp.int32),
          pltpu.VMEM((BPW, DIM), jnp.float32),
          pltpu.SemaphoreType.DMA,
      ],
  )(_sc_comgather_body)
  return kern(user, p_norm)


def kernel(user, pos, neg, mf_user_w, mf_item_w, mlp_user_w, mlp_item_w,
           train_label):
  user = user.astype(jnp.int32)
  pos = pos.astype(jnp.int32)
  neg = neg.astype(jnp.int32)

  t = train_label.T
  w_aug = jnp.concatenate(
      [mf_item_w, mlp_item_w,
       jnp.ones((NUM_ITEMS_P1, 1), jnp.float32),
       jnp.zeros((NUM_ITEMS_P1, NAUG - 257), jnp.float32)],
      axis=1).T.astype(jnp.bfloat16)

  (mf_user_emb, mlp_user_emb, mf_pos_emb, mf_neg_emb, mlp_pos_emb,
   mlp_neg_emb) = _sc_embed(user, pos, neg, mf_user_w, mlp_user_w,
                            mf_item_w, mlp_item_w)
  p_norm = _community_all(t, w_aug)
  pk = _sc_comgather(user, p_norm)
  bits = lax.bitcast_convert_type(pk, jnp.uint32)
  mf_pos_i_com = lax.bitcast_convert_type(
      (bits & 0xFFFF).astype(jnp.uint16), jnp.bfloat16).astype(jnp.float32)
  mlp_pos_i_com = lax.bitcast_convert_type(
      (bits >> 16).astype(jnp.uint16), jnp.bfloat16).astype(jnp.float32)
  return (mf_user_emb, mf_pos_emb, mf_neg_emb, mf_pos_i_com,
          mlp_user_emb, mlp_pos_emb, mlp_neg_emb, mlp_pos_i_com)


# speedup vs baseline: 9.3051x; 1.0048x over previous
import functools

import jax
import jax.numpy as jnp
from jax import lax
from jax.experimental import pallas as pl
from jax.experimental.pallas import tpu as pltpu
from jax.experimental.pallas import tpu_sc as plsc

NUM_USERS = 100000
NUM_ITEMS_P1 = 1001
DIM = 128
B = 4096

NC = 2
NS = 16
NW = NC * NS
BPW = B // NW

BU = 6144
NAUG = 384


def _community_all_body(t_ref, wt_ref, out_ref):
  tb = t_ref[...].astype(jnp.bfloat16)
  p = lax.dot_general(wt_ref[...], tb,
                      (((1,), (0,)), ((), ())),
                      preferred_element_type=jnp.float32)
  num = p[256:257, :]
  pn = p[:256, :] / num
  pn16 = pn.T.astype(jnp.bfloat16)
  lo = lax.bitcast_convert_type(pn16[:, :128], jnp.uint16).astype(jnp.uint32)
  hi = lax.bitcast_convert_type(pn16[:, 128:], jnp.uint16).astype(jnp.uint32)
  out_ref[...] = lax.bitcast_convert_type(lo | (hi << 16), jnp.float32)


def _community_all(t, w_aug):
  grid = (pl.cdiv(NUM_USERS, BU),)
  return pl.pallas_call(
      _community_all_body,
      grid=grid,
      in_specs=[
          pl.BlockSpec((NUM_ITEMS_P1, BU), lambda i: (0, i)),
          pl.BlockSpec((NAUG, NUM_ITEMS_P1), lambda i: (0, 0)),
      ],
      out_specs=pl.BlockSpec((BU, 128), lambda i: (i, 0)),
      out_shape=jax.ShapeDtypeStruct((NUM_USERS, 128), jnp.float32),
      compiler_params=pltpu.CompilerParams(
          vmem_limit_bytes=100 * 1024 * 1024),
  )(t, w_aug)


def _sc_embed_body(user_hbm, pos_hbm, neg_hbm,
                   mf_user_w, mlp_user_w, mf_item_w, mlp_item_w,
                   mf_user_out, mlp_user_out,
                   mf_pos_out, mf_neg_out, mlp_pos_out, mlp_neg_out,
                   idx_u, idx_p, idx_n, embs, sem_emb):
  wid = lax.axis_index("s") * NC + lax.axis_index("c")
  base = wid * BPW

  pltpu.sync_copy(user_hbm.at[pl.ds(base, BPW)], idx_u)
  pltpu.sync_copy(pos_hbm.at[pl.ds(base, BPW)], idx_p)
  pltpu.sync_copy(neg_hbm.at[pl.ds(base, BPW)], idx_n)

  jobs = ((mf_user_w, idx_u, mf_user_out),
          (mlp_user_w, idx_u, mlp_user_out),
          (mf_item_w, idx_p, mf_pos_out),
          (mf_item_w, idx_n, mf_neg_out),
          (mlp_item_w, idx_p, mlp_pos_out),
          (mlp_item_w, idx_n, mlp_neg_out))
  descs = []
  for j, (table, idx, _) in enumerate(jobs):
    d = pltpu.make_async_copy(table.at[idx], embs.at[j], sem_emb)
    d.start()
    descs.append(d)
  for d in descs:
    d.wait()
  for j, (_, _, out) in enumerate(jobs):
    pltpu.sync_copy(embs.at[j], out.at[pl.ds(base, BPW)])


def _sc_embed(user, pos, neg, mf_user_w, mlp_user_w, mf_item_w, mlp_item_w):
  mesh = plsc.VectorSubcoreMesh(core_axis_name="c", subcore_axis_name="s",
                                num_cores=NC, num_subcores=NS)
  f32 = jnp.float32
  kern = functools.partial(
      pl.kernel,
      out_type=[jax.ShapeDtypeStruct((B, DIM), f32) for _ in range(6)],
      mesh=mesh,
      scratch_types=[
          pltpu.VMEM((BPW,), jnp.int32),
          pltpu.VMEM((BPW,), jnp.int32),
          pltpu.VMEM((BPW,), jnp.int32),
          pltpu.VMEM((6, BPW, DIM), f32),
          pltpu.SemaphoreType.DMA,
      ],
  )(_sc_embed_body)
  return kern(user, pos, neg, mf_user_w, mlp_user_w, mf_item_w, mlp_item_w)


def _sc_comgather_body(user_hbm, p_hbm, pk_out, idx_u, rows, sem):
  wid = lax.axis_index("s") * NC + lax.axis_index("c")
  base = wid * BPW
  pltpu.sync_copy(user_hbm.at[pl.ds(base, BPW)], idx_u)
  pltpu.async_copy(p_hbm.at[idx_u], rows, sem).wait()
  pltpu.sync_copy(rows, pk_out.at[pl.ds(base, BPW)])


def _sc_comgather(user, p_norm):
  mesh = plsc.VectorSubcoreMesh(core_axis_name="c", subcore_axis_name="s",
                                num_cores=NC, num_subcores=NS)
  kern = functools.partial(
      pl.kernel,
      out_type=jax.ShapeDtypeStruct((B, DIM), jnp.float32),
      mesh=mesh,
      scratch_types=[
          pltpu.VMEM((BPW,), jnp.int32),
          pltpu.VMEM((BPW, DIM), jnp.float32),
          pltpu.SemaphoreType.DMA,
      ],
  )(_sc_comgather_body)
  return kern(user, p_norm)


def kernel(user, pos, neg, mf_user_w, mf_item_w, mlp_user_w, mlp_item_w,
           train_label):
  user = user.astype(jnp.int32)
  pos = pos.astype(jnp.int32)
  neg = neg.astype(jnp.int32)

  t = train_label.T
  w_aug = jnp.concatenate(
      [mf_item_w, mlp_item_w,
       jnp.ones((NUM_ITEMS_P1, 1), jnp.float32),
       jnp.zeros((NUM_ITEMS_P1, NAUG - 257), jnp.float32)],
      axis=1).T.astype(jnp.bfloat16)

  p_norm = _community_all(t, w_aug)
  (mf_user_emb, mlp_user_emb, mf_pos_emb, mf_neg_emb, mlp_pos_emb,
   mlp_neg_emb) = _sc_embed(user, pos, neg, mf_user_w, mlp_user_w,
                            mf_item_w, mlp_item_w)
  pk = _sc_comgather(user, p_norm)
  bits = lax.bitcast_convert_type(pk, jnp.uint32)
  mf_pos_i_com = lax.bitcast_convert_type(
      (bits & 0xFFFF).astype(jnp.uint16), jnp.bfloat16).astype(jnp.float32)
  mlp_pos_i_com = lax.bitcast_convert_type(
      (bits >> 16).astype(jnp.uint16), jnp.bfloat16).astype(jnp.float32)
  return (mf_user_emb, mf_pos_emb, mf_neg_emb, mf_pos_i_com,
          mlp_user_emb, mlp_pos_emb, mlp_neg_emb, mlp_pos_i_com)
